# Initial kernel scaffold; baseline (speedup 1.0000x reference)
import jax
import jax.numpy as jnp
from jax import lax
from jax.experimental import pallas as pl
from jax.experimental.pallas import tpu as pltpu
from jax.experimental.pallas import tpu_sc as plsc

CUTOFF = 6.0
NUM_RADIAL = 32

NC = 2
NS = 16
NW = NC * NS
L = 16

CK = 128
CKA = 1024


def _make_dist(n, e_pad):
    ew = e_pad // NW
    mesh = plsc.VectorSubcoreMesh(core_axis_name="c", subcore_axis_name="s")

    def body(posT_hbm, src_hbm, dst_hbm, d2_hbm, px, py, pz, sv, dv, ov):
        cid = lax.axis_index("c")
        sid = lax.axis_index("s")
        wid = sid * NC + cid
        pltpu.sync_copy(posT_hbm.at[0], px)
        pltpu.sync_copy(posT_hbm.at[1], py)
        pltpu.sync_copy(posT_hbm.at[2], pz)

        def chunk(c, carry):
            base = wid * ew + c * CKA
            pltpu.sync_copy(src_hbm.at[pl.ds(base, CKA)], sv)
            pltpu.sync_copy(dst_hbm.at[pl.ds(base, CKA)], dv)

            def inner(i, carry2):
                off = i * L
                a = sv[pl.ds(off, L)]
                b = dv[pl.ds(off, L)]
                dx = plsc.load_gather(px, [a]) - plsc.load_gather(px, [b])
                dy = plsc.load_gather(py, [a]) - plsc.load_gather(py, [b])
                dz = plsc.load_gather(pz, [a]) - plsc.load_gather(pz, [b])
                ov[pl.ds(off, L)] = dx * dx + dy * dy + dz * dz
                return carry2

            lax.fori_loop(0, CKA // L, inner, 0)
            pltpu.sync_copy(ov, d2_hbm.at[pl.ds(base, CKA)])
            return carry

        lax.fori_loop(0, ew // CKA, chunk, 0)

    return pl.kernel(
        body,
        out_type=jax.ShapeDtypeStruct((e_pad,), jnp.float32),
        mesh=mesh,
        scratch_types=[
            pltpu.VMEM((n,), jnp.float32),
            pltpu.VMEM((n,), jnp.float32),
            pltpu.VMEM((n,), jnp.float32),
            pltpu.VMEM((CKA,), jnp.int32),
            pltpu.VMEM((CKA,), jnp.int32),
            pltpu.VMEM((CKA,), jnp.float32),
        ],
    )


def _make_msg(n, e_pad, dm):
    ew = e_pad // NW
    nchunk = ew // CK
    rt = n // NS
    mesh = plsc.VectorSubcoreMesh(core_axis_name="c", subcore_axis_name="s")

    def body(hs_hbm, t_hbm, src_hbm, dst_hbm, z_hbm, part_hbm,
             aggs, si, di, tv, gv, bounce, sem):
        cid = lax.axis_index("c")
        sid = lax.axis_index("s")
        wid = sid * NC + cid
        rows = pl.ds(sid * rt, rt)
        pltpu.sync_copy(z_hbm, aggs.at[rows])
        plsc.subcore_barrier()

        def chunk(c, carry):
            base = wid * ew + c * CK
            pltpu.sync_copy(src_hbm.at[pl.ds(base, CK)], si)
            pltpu.sync_copy(dst_hbm.at[pl.ds(base, CK)], di)
            pltpu.async_copy(hs_hbm.at[si], gv, sem).wait()
            pltpu.sync_copy(t_hbm.at[pl.ds(base, CK)], tv)

            def mrow(r, carry2):
                for j in range(dm // L):
                    gv[r, pl.ds(j * L, L)] = (
                        gv[r, pl.ds(j * L, L)] * tv[r, pl.ds(j * L, L)])
                return carry2

            lax.fori_loop(0, CK, mrow, 0)
            pltpu.sync_copy(gv, aggs.at[di], add=True)
            return carry

        lax.fori_loop(0, nchunk, chunk, 0)
        plsc.subcore_barrier()
        pltpu.sync_copy(aggs.at[rows], bounce)
        pltpu.sync_copy(bounce, part_hbm.at[cid, rows])

    return pl.kernel(
        body,
        out_type=jax.ShapeDtypeStruct((NC, n, dm), jnp.float32),
        mesh=mesh,
        scratch_types=[
            pltpu.VMEM_SHARED((n, dm), jnp.float32),
            pltpu.VMEM((CK,), jnp.int32),
            pltpu.VMEM((CK,), jnp.int32),
            pltpu.VMEM((CK, dm), jnp.float32),
            pltpu.VMEM((CK, dm), jnp.float32),
            pltpu.VMEM((n // NS, dm), jnp.float32),
            pltpu.SemaphoreType.DMA,
        ],
    )


def _node_body(x_ref, we_ref, be_ref, ws_ref, h_ref, hs_ref):
    h = jax.nn.gelu(
        jnp.dot(x_ref[...], we_ref[...], preferred_element_type=jnp.float32)
        + be_ref[...])
    h_ref[...] = h
    hs_ref[...] = jnp.dot(h, ws_ref[...], preferred_element_type=jnp.float32)


def _make_rbf_body(e, blk):
    sigma = CUTOFF / NUM_RADIAL
    inv2s2 = 1.0 / (2.0 * sigma * sigma)
    step = CUTOFF / (NUM_RADIAL - 1)

    def body(d2_ref, w_ref, t_ref):
        pid = pl.program_id(0)
        d = jnp.sqrt(d2_ref[...] + 1e-8)
        row = pid * blk + lax.broadcasted_iota(jnp.int32, (blk, 1), 0)
        env = 0.5 * (jnp.cos(jnp.pi * jnp.clip(d / CUTOFF, 0.0, 1.0)) + 1.0)
        env = jnp.where(row < e, env, 0.0)
        centers = lax.broadcasted_iota(jnp.float32, (1, NUM_RADIAL), 1) * step
        rbf = jnp.exp(-((d - centers) ** 2) * inv2s2) * env
        t_ref[...] = jnp.dot(rbf, w_ref[...],
                             preferred_element_type=jnp.float32)

    return body


def _make_upd_body(rows_per_graph, gpb, r):
    def body(h_ref, p_ref, wuh_ref, wua_ref, bu_ref, wo_ref, bo_ref, out_ref):
        agg = p_ref[0] + p_ref[1]
        h2 = jax.nn.gelu(
            jnp.dot(h_ref[...], wuh_ref[...], preferred_element_type=jnp.float32)
            + jnp.dot(agg, wua_ref[...], preferred_element_type=jnp.float32)
            + bu_ref[...])
        ne = jnp.dot(h2, wo_ref[...], preferred_element_type=jnp.float32) + bo_ref[...]
        gi = lax.broadcasted_iota(jnp.int32, (gpb, r), 0)
        ri = lax.broadcasted_iota(jnp.int32, (gpb, r), 1)
        pool = jnp.where(ri // rows_per_graph == gi, 1.0 / rows_per_graph, 0.0)
        out_ref[...] = jnp.dot(pool, ne, preferred_element_type=jnp.float32)

    return body


def _mlp_body(g_ref, w0, b0, w1, b1, w2, b2, w3, b3, y_ref):
    y = jax.nn.gelu(jnp.dot(g_ref[...], w0[...], preferred_element_type=jnp.float32) + b0[...])
    y = jax.nn.gelu(jnp.dot(y, w1[...], preferred_element_type=jnp.float32) + b1[...])
    y = jax.nn.gelu(jnp.dot(y, w2[...], preferred_element_type=jnp.float32) + b2[...])
    y_ref[...] = jax.nn.gelu(jnp.dot(y, w3[...], preferred_element_type=jnp.float32) + b3[...])


def kernel(x, pos, edge_index, batch, ptr,
           W_enc, b_enc, W_src, W_rbf, W_upd, b_upd, W_out, b_out,
           W_fc0, b_fc0, W_fc1, b_fc1, W_fc2, b_fc2, W_fc3, b_fc3):
    n, d_in = x.shape
    e = edge_index.shape[1]
    g = ptr.shape[0] - 1
    d_node = W_enc.shape[1]
    d_msg = W_src.shape[1]
    d_emb = W_out.shape[1]
    fc = W_fc0.shape[1]
    rows_per_graph = n // g

    unit = NW * CKA
    e_pad = ((e + unit - 1) // unit) * unit

    src = edge_index[0]
    dst = edge_index[1]
    padlen = e_pad - e
    if padlen:
        zpad = jnp.zeros((padlen,), jnp.int32)
        src_p = jnp.concatenate([src, zpad])
        dst_p = jnp.concatenate([dst, zpad])
    else:
        src_p, dst_p = src, dst
    posT = jnp.transpose(pos)

    d2 = _make_dist(n, e_pad)(posT, src_p, dst_p)

    rb = 400
    h, hs = pl.pallas_call(
        _node_body,
        grid=(n // rb,),
        in_specs=[
            pl.BlockSpec((rb, d_in), lambda i: (i, 0)),
            pl.BlockSpec((d_in, d_node), lambda i: (0, 0)),
            pl.BlockSpec((1, d_node), lambda i: (0, 0)),
            pl.BlockSpec((d_node, d_msg), lambda i: (0, 0)),
        ],
        out_specs=[
            pl.BlockSpec((rb, d_node), lambda i: (i, 0)),
            pl.BlockSpec((rb, d_msg), lambda i: (i, 0)),
        ],
        out_shape=[
            jax.ShapeDtypeStruct((n, d_node), jnp.float32),
            jax.ShapeDtypeStruct((n, d_msg), jnp.float32),
        ],
    )(x, W_enc, b_enc.reshape(1, d_node), W_src)

    eb = 512
    t = pl.pallas_call(
        _make_rbf_body(e, eb),
        grid=(e_pad // eb,),
        in_specs=[
            pl.BlockSpec((eb, 1), lambda i: (i, 0)),
            pl.BlockSpec((NUM_RADIAL, d_msg), lambda i: (0, 0)),
        ],
        out_specs=pl.BlockSpec((eb, d_msg), lambda i: (i, 0)),
        out_shape=jax.ShapeDtypeStruct((e_pad, d_msg), jnp.float32),
    )(d2.reshape(e_pad, 1), W_rbf)

    zrows = jnp.zeros((n // NS, d_msg), jnp.float32)
    part = _make_msg(n, e_pad, d_msg)(hs, t, src_p, dst_p, zrows)

    ru = 400
    gpb = ru // rows_per_graph
    pooled = pl.pallas_call(
        _make_upd_body(rows_per_graph, gpb, ru),
        grid=(n // ru,),
        in_specs=[
            pl.BlockSpec((ru, d_node), lambda i: (i, 0)),
            pl.BlockSpec((NC, ru, d_msg), lambda i: (0, i, 0)),
            pl.BlockSpec((d_node, d_node), lambda i: (0, 0)),
            pl.BlockSpec((d_msg, d_node), lambda i: (0, 0)),
            pl.BlockSpec((1, d_node), lambda i: (0, 0)),
            pl.BlockSpec((d_node, d_emb), lambda i: (0, 0)),
            pl.BlockSpec((1, d_emb), lambda i: (0, 0)),
        ],
        out_specs=pl.BlockSpec((gpb, d_emb), lambda i: (i, 0)),
        out_shape=jax.ShapeDtypeStruct((g, d_emb), jnp.float32),
    )(h, part, W_upd[:d_node], W_upd[d_node:], b_upd.reshape(1, d_node),
      W_out, b_out.reshape(1, d_emb))

    y = pl.pallas_call(
        _mlp_body,
        out_shape=jax.ShapeDtypeStruct((g, fc), jnp.float32),
    )(pooled, W_fc0, b_fc0.reshape(1, fc), W_fc1, b_fc1.reshape(1, fc),
      W_fc2, b_fc2.reshape(1, fc), W_fc3, b_fc3.reshape(1, fc))
    return y


# speedup vs baseline: 2.0926x; 2.0926x over previous
import jax
import jax.numpy as jnp
from jax import lax
from jax.experimental import pallas as pl
from jax.experimental.pallas import tpu as pltpu
from jax.experimental.pallas import tpu_sc as plsc

CUTOFF = 6.0
NUM_RADIAL = 32

NC = 2
NS = 16
NW = NC * NS
L = 16

CK = 128
CKA = 1024


def _make_dist(n, e_pad):
    ew = e_pad // NW
    mesh = plsc.VectorSubcoreMesh(core_axis_name="c", subcore_axis_name="s")

    def body(px_hbm, py_hbm, pz_hbm, src_hbm, dst_hbm, d2_hbm,
             px, py, pz, sv, dv, ov):
        cid = lax.axis_index("c")
        sid = lax.axis_index("s")
        wid = sid * NC + cid
        pltpu.sync_copy(px_hbm, px)
        pltpu.sync_copy(py_hbm, py)
        pltpu.sync_copy(pz_hbm, pz)

        def chunk(c, carry):
            base = wid * ew + c * CKA
            pltpu.sync_copy(src_hbm.at[pl.ds(base, CKA)], sv)
            pltpu.sync_copy(dst_hbm.at[pl.ds(base, CKA)], dv)

            def inner(i, carry2):
                off = i * L
                a = sv[pl.ds(off, L)]
                b = dv[pl.ds(off, L)]
                dx = plsc.load_gather(px, [a]) - plsc.load_gather(px, [b])
                dy = plsc.load_gather(py, [a]) - plsc.load_gather(py, [b])
                dz = plsc.load_gather(pz, [a]) - plsc.load_gather(pz, [b])
                ov[pl.ds(off, L)] = dx * dx + dy * dy + dz * dz
                return carry2

            lax.fori_loop(0, CKA // L, inner, 0)
            pltpu.sync_copy(ov, d2_hbm.at[pl.ds(base, CKA)])
            return carry

        lax.fori_loop(0, ew // CKA, chunk, 0)

    return pl.kernel(
        body,
        out_type=jax.ShapeDtypeStruct((e_pad,), jnp.float32),
        mesh=mesh,
        scratch_types=[
            pltpu.VMEM((n,), jnp.float32),
            pltpu.VMEM((n,), jnp.float32),
            pltpu.VMEM((n,), jnp.float32),
            pltpu.VMEM((CKA,), jnp.int32),
            pltpu.VMEM((CKA,), jnp.int32),
            pltpu.VMEM((CKA,), jnp.float32),
        ],
        compiler_params=pltpu.CompilerParams(needs_layout_passes=False),
    )


def _make_msg(n_pad, e_pad, dm):
    ew = e_pad // NW
    nchunk = ew // CK
    rt = n_pad // NS
    mesh = plsc.VectorSubcoreMesh(core_axis_name="c", subcore_axis_name="s")

    def body(hs_hbm, t_hbm, src_hbm, dst_hbm, z_hbm, part_hbm,
             aggs, si, di, tv, gv, bounce, sem):
        cid = lax.axis_index("c")
        sid = lax.axis_index("s")
        wid = sid * NC + cid
        rows = pl.ds(sid * rt, rt)
        pltpu.sync_copy(z_hbm, aggs.at[rows])
        plsc.subcore_barrier()

        def chunk(c, carry):
            base = wid * ew + c * CK
            pltpu.sync_copy(src_hbm.at[pl.ds(base, CK)], si)
            pltpu.sync_copy(dst_hbm.at[pl.ds(base, CK)], di)
            pltpu.async_copy(hs_hbm.at[si], gv, sem).wait()
            pltpu.sync_copy(t_hbm.at[pl.ds(base, CK)], tv)

            def mrow(r, carry2):
                for j in range(dm // L):
                    gv[r, pl.ds(j * L, L)] = (
                        gv[r, pl.ds(j * L, L)] * tv[r, pl.ds(j * L, L)])
                return carry2

            lax.fori_loop(0, CK, mrow, 0)
            pltpu.sync_copy(gv, aggs.at[di], add=True)
            return carry

        lax.fori_loop(0, nchunk, chunk, 0)
        plsc.subcore_barrier()
        pltpu.sync_copy(aggs.at[rows], bounce)
        pltpu.sync_copy(bounce, part_hbm.at[cid, rows])

    return pl.kernel(
        body,
        out_type=jax.ShapeDtypeStruct((NC, n_pad, dm), jnp.float32),
        mesh=mesh,
        scratch_types=[
            pltpu.VMEM_SHARED((n_pad, dm), jnp.float32),
            pltpu.VMEM((CK,), jnp.int32),
            pltpu.VMEM((CK,), jnp.int32),
            pltpu.VMEM((CK, dm), jnp.float32),
            pltpu.VMEM((CK, dm), jnp.float32),
            pltpu.VMEM((rt, dm), jnp.float32),
            pltpu.SemaphoreType.DMA,
        ],
        compiler_params=pltpu.CompilerParams(
            needs_layout_passes=False, use_tc_tiling_on_sc=False),
    )


def _node_body(x_ref, we_ref, be_ref, ws_ref, h_ref, hs_ref):
    h = jax.nn.gelu(
        jnp.dot(x_ref[...], we_ref[...], preferred_element_type=jnp.float32)
        + be_ref[...])
    h_ref[...] = h
    hs_ref[...] = jnp.dot(h, ws_ref[...], preferred_element_type=jnp.float32)


def _make_rbf_body(e, blk):
    sigma = CUTOFF / NUM_RADIAL
    inv2s2 = 1.0 / (2.0 * sigma * sigma)
    step = CUTOFF / (NUM_RADIAL - 1)

    def body(d2_ref, w_ref, t_ref):
        pid = pl.program_id(0)
        d = jnp.sqrt(d2_ref[...] + 1e-8)
        row = pid * blk + lax.broadcasted_iota(jnp.int32, (blk, 1), 0)
        env = 0.5 * (jnp.cos(jnp.pi * jnp.clip(d / CUTOFF, 0.0, 1.0)) + 1.0)
        env = jnp.where(row < e, env, 0.0)
        centers = lax.broadcasted_iota(
            jnp.int32, (1, NUM_RADIAL), 1).astype(jnp.float32) * step
        rbf = jnp.exp(-((d - centers) ** 2) * inv2s2) * env
        t_ref[...] = jnp.dot(rbf, w_ref[...],
                             preferred_element_type=jnp.float32)

    return body


def _make_upd_body(rows_per_graph, gpb, r):
    def body(h_ref, p_ref, wuh_ref, wua_ref, bu_ref, wo_ref, bo_ref, out_ref):
        agg = p_ref[0] + p_ref[1]
        h2 = jax.nn.gelu(
            jnp.dot(h_ref[...], wuh_ref[...], preferred_element_type=jnp.float32)
            + jnp.dot(agg, wua_ref[...], preferred_element_type=jnp.float32)
            + bu_ref[...])
        ne = jnp.dot(h2, wo_ref[...], preferred_element_type=jnp.float32) + bo_ref[...]
        gi = lax.broadcasted_iota(jnp.int32, (gpb, r), 0)
        ri = lax.broadcasted_iota(jnp.int32, (gpb, r), 1)
        pool = jnp.where(ri // rows_per_graph == gi, 1.0 / rows_per_graph, 0.0)
        out_ref[0] = jnp.dot(pool, ne, preferred_element_type=jnp.float32)

    return body


def _mlp_body(g_ref, w0, b0, w1, b1, w2, b2, w3, b3, y_ref):
    y = jax.nn.gelu(jnp.dot(g_ref[...], w0[...], preferred_element_type=jnp.float32) + b0[...])
    y = jax.nn.gelu(jnp.dot(y, w1[...], preferred_element_type=jnp.float32) + b1[...])
    y = jax.nn.gelu(jnp.dot(y, w2[...], preferred_element_type=jnp.float32) + b2[...])
    y_ref[...] = jax.nn.gelu(jnp.dot(y, w3[...], preferred_element_type=jnp.float32) + b3[...])


def kernel(x, pos, edge_index, batch, ptr,
           W_enc, b_enc, W_src, W_rbf, W_upd, b_upd, W_out, b_out,
           W_fc0, b_fc0, W_fc1, b_fc1, W_fc2, b_fc2, W_fc3, b_fc3):
    n, d_in = x.shape
    e = edge_index.shape[1]
    g = ptr.shape[0] - 1
    d_node = W_enc.shape[1]
    d_msg = W_src.shape[1]
    d_emb = W_out.shape[1]
    fc = W_fc0.shape[1]
    rows_per_graph = n // g

    unit = NW * CKA
    e_pad = ((e + unit - 1) // unit) * unit

    src = edge_index[0]
    dst = edge_index[1]
    padlen = e_pad - e
    if padlen:
        zpad = jnp.zeros((padlen,), jnp.int32)
        src_p = jnp.concatenate([src, zpad])
        dst_p = jnp.concatenate([dst, zpad])
    else:
        src_p, dst_p = src, dst
    posT = jnp.transpose(pos)

    d2 = _make_dist(n, e_pad)(posT[0], posT[1], posT[2], src_p, dst_p)

    rb = 400
    h, hs = pl.pallas_call(
        _node_body,
        grid=(n // rb,),
        in_specs=[
            pl.BlockSpec((rb, d_in), lambda i: (i, 0)),
            pl.BlockSpec((d_in, d_node), lambda i: (0, 0)),
            pl.BlockSpec((1, d_node), lambda i: (0, 0)),
            pl.BlockSpec((d_node, d_msg), lambda i: (0, 0)),
        ],
        out_specs=[
            pl.BlockSpec((rb, d_node), lambda i: (i, 0)),
            pl.BlockSpec((rb, d_msg), lambda i: (i, 0)),
        ],
        out_shape=[
            jax.ShapeDtypeStruct((n, d_node), jnp.float32),
            jax.ShapeDtypeStruct((n, d_msg), jnp.float32),
        ],
    )(x, W_enc, b_enc.reshape(1, d_node), W_src)

    eb = 512
    t = pl.pallas_call(
        _make_rbf_body(e, eb),
        grid=(e_pad // eb,),
        in_specs=[
            pl.BlockSpec((eb, 1), lambda i: (i, 0)),
            pl.BlockSpec((NUM_RADIAL, d_msg), lambda i: (0, 0)),
        ],
        out_specs=pl.BlockSpec((eb, d_msg), lambda i: (i, 0)),
        out_shape=jax.ShapeDtypeStruct((e_pad, d_msg), jnp.float32),
    )(d2.reshape(e_pad, 1), W_rbf)

    n_pad = ((n + NS * 8 - 1) // (NS * 8)) * (NS * 8)
    zrows = jnp.zeros((n_pad // NS, d_msg), jnp.float32)
    part = _make_msg(n_pad, e_pad, d_msg)(hs, t, src_p, dst_p, zrows)

    ru = 400
    gpb = ru // rows_per_graph
    pooled = pl.pallas_call(
        _make_upd_body(rows_per_graph, gpb, ru),
        grid=(n // ru,),
        in_specs=[
            pl.BlockSpec((ru, d_node), lambda i: (i, 0)),
            pl.BlockSpec((NC, ru, d_msg), lambda i: (0, i, 0)),
            pl.BlockSpec((d_node, d_node), lambda i: (0, 0)),
            pl.BlockSpec((d_msg, d_node), lambda i: (0, 0)),
            pl.BlockSpec((1, d_node), lambda i: (0, 0)),
            pl.BlockSpec((d_node, d_emb), lambda i: (0, 0)),
            pl.BlockSpec((1, d_emb), lambda i: (0, 0)),
        ],
        out_specs=pl.BlockSpec((1, gpb, d_emb), lambda i: (i, 0, 0)),
        out_shape=jax.ShapeDtypeStruct((n // ru, gpb, d_emb), jnp.float32),
    )(h, part, W_upd[:d_node], W_upd[d_node:], b_upd.reshape(1, d_node),
      W_out, b_out.reshape(1, d_emb))
    pooled = pooled.reshape(g, d_emb)

    y = pl.pallas_call(
        _mlp_body,
        out_shape=jax.ShapeDtypeStruct((g, fc), jnp.float32),
    )(pooled, W_fc0, b_fc0.reshape(1, fc), W_fc1, b_fc1.reshape(1, fc),
      W_fc2, b_fc2.reshape(1, fc), W_fc3, b_fc3.reshape(1, fc))
    return y


# speedup vs baseline: 5.3383x; 2.5511x over previous
import jax
import jax.numpy as jnp
from jax import lax
from jax.experimental import pallas as pl
from jax.experimental.pallas import tpu as pltpu
from jax.experimental.pallas import tpu_sc as plsc

CUTOFF = 6.0
NUM_RADIAL = 32

NC = 2
NS = 16
NW = NC * NS
L = 16

CK = 128
CKA = 1024


def _make_dist(n, e_pad):
    ew = e_pad // NW
    mesh = plsc.VectorSubcoreMesh(core_axis_name="c", subcore_axis_name="s")

    def body(px_hbm, py_hbm, pz_hbm, src_hbm, dst_hbm, d2_hbm,
             px, py, pz, sv, dv, ov):
        cid = lax.axis_index("c")
        sid = lax.axis_index("s")
        wid = sid * NC + cid
        pltpu.sync_copy(px_hbm, px)
        pltpu.sync_copy(py_hbm, py)
        pltpu.sync_copy(pz_hbm, pz)

        def chunk(c, carry):
            base = wid * ew + c * CKA
            pltpu.sync_copy(src_hbm.at[pl.ds(base, CKA)], sv)
            pltpu.sync_copy(dst_hbm.at[pl.ds(base, CKA)], dv)

            def inner(i, carry2):
                off = i * L
                a = sv[pl.ds(off, L)]
                b = dv[pl.ds(off, L)]
                dx = plsc.load_gather(px, [a]) - plsc.load_gather(px, [b])
                dy = plsc.load_gather(py, [a]) - plsc.load_gather(py, [b])
                dz = plsc.load_gather(pz, [a]) - plsc.load_gather(pz, [b])
                ov[pl.ds(off, L)] = dx * dx + dy * dy + dz * dz
                return carry2

            lax.fori_loop(0, CKA // L, inner, 0)
            pltpu.sync_copy(ov, d2_hbm.at[pl.ds(base, CKA)])
            return carry

        lax.fori_loop(0, ew // CKA, chunk, 0)

    return pl.kernel(
        body,
        out_type=jax.ShapeDtypeStruct((e_pad,), jnp.float32),
        mesh=mesh,
        scratch_types=[
            pltpu.VMEM((n,), jnp.float32),
            pltpu.VMEM((n,), jnp.float32),
            pltpu.VMEM((n,), jnp.float32),
            pltpu.VMEM((CKA,), jnp.int32),
            pltpu.VMEM((CKA,), jnp.int32),
            pltpu.VMEM((CKA,), jnp.float32),
        ],
        compiler_params=pltpu.CompilerParams(needs_layout_passes=False),
    )


def _make_msg(n_pad, e_pad, dm):
    ew = e_pad // NW
    nchunk = ew // CK
    rt = n_pad // NS
    mesh = plsc.VectorSubcoreMesh(core_axis_name="c", subcore_axis_name="s")

    def body(hs_hbm, t_hbm, src2_hbm, dst2_hbm, z_hbm, part_hbm,
             aggs, siv, div, gv0, gv1, tv0, tv1, mv0, mv1,
             sg0, sg1, st0, st1, ss0, ss1):
        gv = (gv0, gv1)
        tv = (tv0, tv1)
        mv = (mv0, mv1)
        sg = (sg0, sg1)
        st = (st0, st1)
        ss = (ss0, ss1)
        cid = lax.axis_index("c")
        sid = lax.axis_index("s")
        wid = sid * NC + cid
        rows = pl.ds(sid * rt, rt)
        pltpu.sync_copy(z_hbm, aggs.at[rows])
        pltpu.sync_copy(src2_hbm.at[pl.ds(wid * nchunk, nchunk)], siv)
        pltpu.sync_copy(dst2_hbm.at[pl.ds(wid * nchunk, nchunk)], div)

        def start_in(c, b):
            pltpu.async_copy(hs_hbm.at[siv.at[c]], gv[b], sg[b])
            pltpu.async_copy(t_hbm.at[pl.ds(wid * ew + c * CK, CK)],
                             tv[b], st[b])

        plsc.subcore_barrier()
        for b in range(2):
            start_in(b, b)

        def outer(c2, carry):
            for b in range(2):
                c = c2 * 2 + b
                pltpu.make_async_copy(hs_hbm.at[siv.at[c]], gv[b],
                                      sg[b]).wait()
                pltpu.make_async_copy(
                    t_hbm.at[pl.ds(wid * ew + c * CK, CK)], tv[b],
                    st[b]).wait()

                @pl.when(c2 > 0)
                def _():
                    pltpu.make_async_copy(mv[b], aggs.at[div.at[c]],
                                          ss[b]).wait()

                @plsc.parallel_loop(0, CK, step=1, unroll=8)
                def mrow(r):
                    for j in range(dm // L):
                        mv[b][r, pl.ds(j * L, L)] = (
                            gv[b][r, pl.ds(j * L, L)]
                            * tv[b][r, pl.ds(j * L, L)])

                pltpu.async_copy(mv[b], aggs.at[div.at[c]], ss[b], add=True)

                @pl.when(c2 < nchunk // 2 - 1)
                def _():
                    start_in(c + 2, b)
            return carry

        lax.fori_loop(0, nchunk // 2, outer, 0)
        for b in range(2):
            pltpu.make_async_copy(mv[b], aggs.at[div.at[b]], ss[b]).wait()
        plsc.subcore_barrier()
        pltpu.sync_copy(aggs.at[rows], part_hbm.at[cid, rows])

    return pl.kernel(
        body,
        out_type=jax.ShapeDtypeStruct((NC, n_pad, dm), jnp.float32),
        mesh=mesh,
        scratch_types=[
            pltpu.VMEM_SHARED((n_pad, dm), jnp.float32),
            pltpu.VMEM((nchunk, CK), jnp.int32),
            pltpu.VMEM((nchunk, CK), jnp.int32),
            pltpu.VMEM((CK, dm), jnp.float32),
            pltpu.VMEM((CK, dm), jnp.float32),
            pltpu.VMEM((CK, dm), jnp.float32),
            pltpu.VMEM((CK, dm), jnp.float32),
            pltpu.VMEM((CK, dm), jnp.float32),
            pltpu.VMEM((CK, dm), jnp.float32),
            pltpu.SemaphoreType.DMA,
            pltpu.SemaphoreType.DMA,
            pltpu.SemaphoreType.DMA,
            pltpu.SemaphoreType.DMA,
            pltpu.SemaphoreType.DMA,
            pltpu.SemaphoreType.DMA,
        ],
        compiler_params=pltpu.CompilerParams(
            needs_layout_passes=False, use_tc_tiling_on_sc=False),
    )


def _node_body(x_ref, we_ref, be_ref, ws_ref, h_ref, hs_ref):
    h = jax.nn.gelu(
        jnp.dot(x_ref[...], we_ref[...], preferred_element_type=jnp.float32)
        + be_ref[...])
    h_ref[...] = h
    hs_ref[...] = jnp.dot(h, ws_ref[...], preferred_element_type=jnp.float32)


def _make_rbf_body(e, lanes, rows):
    sigma = CUTOFF / NUM_RADIAL
    inv2s2 = 1.0 / (2.0 * sigma * sigma)
    step = CUTOFF / (NUM_RADIAL - 1)

    def body(d2_ref, wt_ref, t_ref):
        pid = pl.program_id(0)
        d2 = d2_ref[...]
        d = jnp.sqrt(d2 + 1e-8)
        env = 0.5 * (jnp.cos(jnp.pi * jnp.clip(d / CUTOFF, 0.0, 1.0)) + 1.0)
        eid = (pid * (rows * lanes)
               + lax.broadcasted_iota(jnp.int32, (rows, lanes), 0) * lanes
               + lax.broadcasted_iota(jnp.int32, (rows, lanes), 1))
        env = jnp.where(eid < e, env, 0.0)
        centers = lax.broadcasted_iota(
            jnp.int32, (NUM_RADIAL, 1), 0).astype(jnp.float32) * step
        wt = wt_ref[...]
        for r in range(rows):
            dr = lax.slice(d, (r, 0), (r + 1, lanes))
            er = lax.slice(env, (r, 0), (r + 1, lanes))
            db = jnp.broadcast_to(dr, (NUM_RADIAL, lanes))
            rbf_t = jnp.exp(-((db - centers) ** 2) * inv2s2)
            t_t = jnp.dot(wt, rbf_t, preferred_element_type=jnp.float32)
            t_t = t_t * er
            t_ref[pl.ds(r * lanes, lanes), :] = t_t.T

    return body


def _make_upd_body(rows_per_graph, gpb, r):
    def body(h_ref, p_ref, wuh_ref, wua_ref, bu_ref, wo_ref, bo_ref, out_ref):
        agg = p_ref[0] + p_ref[1]
        h2 = jax.nn.gelu(
            jnp.dot(h_ref[...], wuh_ref[...], preferred_element_type=jnp.float32)
            + jnp.dot(agg, wua_ref[...], preferred_element_type=jnp.float32)
            + bu_ref[...])
        ne = jnp.dot(h2, wo_ref[...], preferred_element_type=jnp.float32) + bo_ref[...]
        gi = lax.broadcasted_iota(jnp.int32, (gpb, r), 0)
        ri = lax.broadcasted_iota(jnp.int32, (gpb, r), 1)
        pool = jnp.where(ri // rows_per_graph == gi, 1.0 / rows_per_graph, 0.0)
        out_ref[0] = jnp.dot(pool, ne, preferred_element_type=jnp.float32)

    return body


def _mlp_body(g_ref, w0, b0, w1, b1, w2, b2, w3, b3, y_ref):
    y = jax.nn.gelu(jnp.dot(g_ref[...], w0[...], preferred_element_type=jnp.float32) + b0[...])
    y = jax.nn.gelu(jnp.dot(y, w1[...], preferred_element_type=jnp.float32) + b1[...])
    y = jax.nn.gelu(jnp.dot(y, w2[...], preferred_element_type=jnp.float32) + b2[...])
    y_ref[...] = jax.nn.gelu(jnp.dot(y, w3[...], preferred_element_type=jnp.float32) + b3[...])


def kernel(x, pos, edge_index, batch, ptr,
           W_enc, b_enc, W_src, W_rbf, W_upd, b_upd, W_out, b_out,
           W_fc0, b_fc0, W_fc1, b_fc1, W_fc2, b_fc2, W_fc3, b_fc3):
    n, d_in = x.shape
    e = edge_index.shape[1]
    g = ptr.shape[0] - 1
    d_node = W_enc.shape[1]
    d_msg = W_src.shape[1]
    d_emb = W_out.shape[1]
    fc = W_fc0.shape[1]
    rows_per_graph = n // g

    unit = NW * CKA
    e_pad = ((e + unit - 1) // unit) * unit

    src = edge_index[0]
    dst = edge_index[1]
    padlen = e_pad - e
    if padlen:
        zpad = jnp.zeros((padlen,), jnp.int32)
        src_p = jnp.concatenate([src, zpad])
        dst_p = jnp.concatenate([dst, zpad])
    else:
        src_p, dst_p = src, dst
    posT = jnp.transpose(pos)

    d2 = _make_dist(n, e_pad)(posT[0], posT[1], posT[2], src_p, dst_p)

    rb = 400
    h, hs = pl.pallas_call(
        _node_body,
        grid=(n // rb,),
        in_specs=[
            pl.BlockSpec((rb, d_in), lambda i: (i, 0)),
            pl.BlockSpec((d_in, d_node), lambda i: (0, 0)),
            pl.BlockSpec((1, d_node), lambda i: (0, 0)),
            pl.BlockSpec((d_node, d_msg), lambda i: (0, 0)),
        ],
        out_specs=[
            pl.BlockSpec((rb, d_node), lambda i: (i, 0)),
            pl.BlockSpec((rb, d_msg), lambda i: (i, 0)),
        ],
        out_shape=[
            jax.ShapeDtypeStruct((n, d_node), jnp.float32),
            jax.ShapeDtypeStruct((n, d_msg), jnp.float32),
        ],
    )(x, W_enc, b_enc.reshape(1, d_node), W_src)

    lanes, rows = 512, 8
    eb = lanes * rows
    t = pl.pallas_call(
        _make_rbf_body(e, lanes, rows),
        grid=(e_pad // eb,),
        in_specs=[
            pl.BlockSpec((rows, lanes), lambda i: (i, 0)),
            pl.BlockSpec((d_msg, NUM_RADIAL), lambda i: (0, 0)),
        ],
        out_specs=pl.BlockSpec((eb, d_msg), lambda i: (i, 0)),
        out_shape=jax.ShapeDtypeStruct((e_pad, d_msg), jnp.float32),
    )(d2.reshape(e_pad // lanes, lanes), jnp.transpose(W_rbf))

    n_pad = ((n + NS * 8 - 1) // (NS * 8)) * (NS * 8)
    zrows = jnp.zeros((n_pad // NS, d_msg), jnp.float32)
    part = _make_msg(n_pad, e_pad, d_msg)(
        hs, t, src_p.reshape(e_pad // CK, CK), dst_p.reshape(e_pad // CK, CK),
        zrows)

    ru = 400
    gpb = ru // rows_per_graph
    pooled = pl.pallas_call(
        _make_upd_body(rows_per_graph, gpb, ru),
        grid=(n // ru,),
        in_specs=[
            pl.BlockSpec((ru, d_node), lambda i: (i, 0)),
            pl.BlockSpec((NC, ru, d_msg), lambda i: (0, i, 0)),
            pl.BlockSpec((d_node, d_node), lambda i: (0, 0)),
            pl.BlockSpec((d_msg, d_node), lambda i: (0, 0)),
            pl.BlockSpec((1, d_node), lambda i: (0, 0)),
            pl.BlockSpec((d_node, d_emb), lambda i: (0, 0)),
            pl.BlockSpec((1, d_emb), lambda i: (0, 0)),
        ],
        out_specs=pl.BlockSpec((1, gpb, d_emb), lambda i: (i, 0, 0)),
        out_shape=jax.ShapeDtypeStruct((n // ru, gpb, d_emb), jnp.float32),
    )(h, part, W_upd[:d_node], W_upd[d_node:], b_upd.reshape(1, d_node),
      W_out, b_out.reshape(1, d_emb))
    pooled = pooled.reshape(g, d_emb)

    y = pl.pallas_call(
        _mlp_body,
        out_shape=jax.ShapeDtypeStruct((g, fc), jnp.float32),
    )(pooled, W_fc0, b_fc0.reshape(1, fc), W_fc1, b_fc1.reshape(1, fc),
      W_fc2, b_fc2.reshape(1, fc), W_fc3, b_fc3.reshape(1, fc))
    return y
